# Initial kernel scaffold; baseline (speedup 1.0000x reference)
import jax
import jax.numpy as jnp
from jax.experimental import pallas as pl


def kernel(hidden_states, gate_W, Wg, Wu, Wd, sWg, sWu, sWd):
    raise NotImplementedError("write your pallas kernel here")


# speedup vs baseline: 1.1451x; 1.1451x over previous
import functools

import jax
import jax.numpy as jnp
from jax.experimental import pallas as pl
from jax.experimental.pallas import tpu as pltpu
from jax.experimental.pallas import tpu_sc as plsc

N = 2048
D = 1024
II = 2816
E = 8
T = 256
NT_SH = N // T
NT_EXP = 23
NT = NT_SH + NT_EXP
NTOT = NT * T
RS = 4
DS = D // RS
SC_WIN = 128


def _router_kernel(x_ref, gw_ref, slots_ref, meta_ref, wts_ref, aux_ref):
    x = x_ref[...]
    gw = gw_ref[...]
    logits = jax.lax.dot_general(
        x, gw, (((1,), (1,)), ((), ())), preferred_element_type=jnp.float32)
    ids = jax.lax.broadcasted_iota(jnp.int32, (N, E), 1).astype(jnp.float32)
    m1 = jnp.max(logits, axis=1, keepdims=True)
    i1 = jnp.min(jnp.where(logits == m1, ids, float(E)), axis=1, keepdims=True)
    masked = jnp.where(ids == i1, -jnp.inf, logits)
    m2 = jnp.max(masked, axis=1, keepdims=True)
    i2 = jnp.min(jnp.where(masked == m2, ids, float(E)), axis=1, keepdims=True)
    e2 = jnp.exp(m2 - m1)
    w1 = 1.0 / (1.0 + e2)
    wts_ref[...] = jnp.concatenate([w1, e2 * w1], axis=1)
    p = jnp.exp(logits - m1)
    probs = p / jnp.sum(p, axis=1, keepdims=True)
    meanprob = jnp.mean(probs, axis=0, keepdims=True)
    oh1 = (ids == i1).astype(jnp.float32)
    oh2 = (ids == i2).astype(jnp.float32)
    counts = jnp.sum(oh1 + oh2, axis=0, keepdims=True)
    aux_ref[...] = (0.01 * E / N) * jnp.sum(counts * meanprob).reshape(1, 1)
    m = jnp.concatenate([oh1, oh2], axis=0)
    a = m
    s = 1
    while s < 2 * N:
        a = a + jnp.concatenate(
            [jnp.zeros((s, E), jnp.float32), a[: 2 * N - s]], axis=0)
        s *= 2
    r = jnp.sum((a - m) * m, axis=1, keepdims=True)
    tiles = jnp.floor((counts + float(T - 1)) / float(T))
    ct = tiles
    s = 1
    while s < E:
        ct = ct + jnp.concatenate(
            [jnp.zeros((1, s), jnp.float32), ct[:, : E - s]], axis=1)
        s *= 2
    base = float(N) + float(T) * (ct - tiles)
    slot = jnp.sum(m * base, axis=1, keepdims=True) + r
    slots_ref[...] = slot.astype(jnp.int32)
    jvec = jax.lax.broadcasted_iota(
        jnp.int32, (NT + 1, 1), 0).astype(jnp.float32)
    jrel = jvec - float(NT_SH)
    done = jnp.sum((jrel >= ct).astype(jnp.float32), axis=1, keepdims=True)
    e_tile = jnp.minimum(done, float(E - 1))
    tile_e = jnp.where(jvec < float(NT_SH), float(E), e_tile)
    nt_used = float(NT_SH) + jnp.sum(tiles)
    meta_ref[...] = jnp.where(
        jvec == float(NT), nt_used, tile_e).astype(jnp.int32)


def _router(x, gate_w):
    return pl.pallas_call(
        _router_kernel,
        out_shape=[
            jax.ShapeDtypeStruct((2 * N, 1), jnp.int32),
            jax.ShapeDtypeStruct((NT + 1, 1), jnp.int32),
            jax.ShapeDtypeStruct((N, 2), jnp.float32),
            jax.ShapeDtypeStruct((1, 1), jnp.float32),
        ],
    )(x, gate_w)


def _dispatch(x4, idx):
    mesh = plsc.VectorSubcoreMesh(core_axis_name="c", subcore_axis_name="s")
    nwin = 3 * N * RS // SC_WIN
    nblk = N * RS // SC_WIN

    @functools.partial(
        pl.kernel,
        out_type=jax.ShapeDtypeStruct((NTOT * RS, DS), jnp.float32),
        mesh=mesh,
        scratch_types=[],
    )
    def k(x_hbm, i_hbm, o_hbm):
        def body(x_vmem, i_vmem):
            pltpu.sync_copy(x_vmem, o_hbm.at[i_vmem.at[0]])

        pltpu.emit_pipeline(
            body,
            grid=(nwin,),
            in_specs=[
                pl.BlockSpec((SC_WIN, DS), index_map=lambda i: (i % nblk, 0)),
                pl.BlockSpec((1, SC_WIN), index_map=lambda i: (i, 0)),
            ],
            out_specs=[],
            core_axis_name=("c", "s"),
            dimension_semantics=(pltpu.PARALLEL,),
        )(x_hbm, i_hbm)

    return k(x4, idx)


def _mlp_kernel(meta_ref, xs_ref, wg_ref, wu_ref, wd_ref, out_ref):
    j = pl.program_id(0)
    nt_used = meta_ref[NT]

    @pl.when(j < nt_used)
    def _():
        x = xs_ref[...].astype(jnp.bfloat16)
        wg = wg_ref[0]
        wu = wu_ref[0]
        wd = wd_ref[0]
        g = jax.lax.dot_general(
            x, wg, (((1,), (1,)), ((), ())), preferred_element_type=jnp.float32)
        u = jax.lax.dot_general(
            x, wu, (((1,), (1,)), ((), ())), preferred_element_type=jnp.float32)
        h = (g * jax.lax.logistic(g) * u).astype(jnp.bfloat16)
        out_ref[...] = jax.lax.dot_general(
            h, wd, (((1,), (1,)), ((), ())), preferred_element_type=jnp.float32)


def _grouped_mlp(meta, xs, w9g, w9u, w9d):
    grid_spec = pltpu.PrefetchScalarGridSpec(
        num_scalar_prefetch=1,
        grid=(NT,),
        in_specs=[
            pl.BlockSpec((T, D), lambda j, meta: (j, 0)),
            pl.BlockSpec((1, II, D), lambda j, meta: (meta[j], 0, 0)),
            pl.BlockSpec((1, II, D), lambda j, meta: (meta[j], 0, 0)),
            pl.BlockSpec((1, D, II), lambda j, meta: (meta[j], 0, 0)),
        ],
        out_specs=pl.BlockSpec((T, D), lambda j, meta: (j, 0)),
    )
    return pl.pallas_call(
        _mlp_kernel,
        grid_spec=grid_spec,
        out_shape=jax.ShapeDtypeStruct((NTOT, D), jnp.float32),
    )(meta, xs, w9g, w9u, w9d)


def _combine_gather(ys4, idx):
    mesh = plsc.VectorSubcoreMesh(core_axis_name="c", subcore_axis_name="s")
    nwin = 3 * N * RS // SC_WIN

    @functools.partial(
        pl.kernel,
        out_type=jax.ShapeDtypeStruct((3 * N * RS, DS), jnp.float32),
        mesh=mesh,
        scratch_types=[],
    )
    def k(y_hbm, i_hbm, o_hbm):
        def body(i_vmem, o_vmem):
            pltpu.sync_copy(y_hbm.at[i_vmem.at[0]], o_vmem)

        pltpu.emit_pipeline(
            body,
            grid=(nwin,),
            in_specs=[pl.BlockSpec((1, SC_WIN), index_map=lambda i: (i, 0))],
            out_specs=[pl.BlockSpec((SC_WIN, DS), index_map=lambda i: (i, 0))],
            core_axis_name=("c", "s"),
            dimension_semantics=(pltpu.PARALLEL,),
        )(i_hbm, o_hbm)

    return k(ys4, idx)


def _combine_kernel(y0_ref, y1_ref, ysh_ref, w_ref, o_ref):
    w = w_ref[...]
    o_ref[...] = (y0_ref[...] * w[:, 0:1] + y1_ref[...] * w[:, 1:2]
                  + ysh_ref[...])


def _combine(y3, wts):
    return pl.pallas_call(
        _combine_kernel,
        grid=(N // T,),
        in_specs=[
            pl.BlockSpec((T, D), lambda i: (i, 0)),
            pl.BlockSpec((T, D), lambda i: (i + N // T, 0)),
            pl.BlockSpec((T, D), lambda i: (i + 2 * (N // T), 0)),
            pl.BlockSpec((T, 2), lambda i: (i, 0)),
        ],
        out_specs=pl.BlockSpec((T, D), lambda i: (i, 0)),
        out_shape=jax.ShapeDtypeStruct((N, D), jnp.float32),
    )(y3, y3, y3, wts)


def kernel(hidden_states, gate_W, Wg, Wu, Wd, sWg, sWu, sWd):
    b, l, d = hidden_states.shape
    x = hidden_states.reshape(l, d)
    w9g = jnp.concatenate([Wg, sWg[None]], axis=0).astype(jnp.bfloat16)
    w9u = jnp.concatenate([Wu, sWu[None]], axis=0).astype(jnp.bfloat16)
    w9d = jnp.concatenate([Wd, sWd[None]], axis=0).astype(jnp.bfloat16)
    slots, meta, wts, aux = _router(x, gate_W)
    idx6 = jnp.concatenate(
        [slots.reshape(-1), jnp.arange(N, dtype=jnp.int32)])
    idx = (idx6[:, None] * RS
           + jnp.arange(RS, dtype=jnp.int32)[None, :]).reshape(-1, SC_WIN)
    xs4 = _dispatch(x.reshape(N * RS, DS), idx)
    ys = _grouped_mlp(meta.reshape(-1), xs4.reshape(NTOT, D), w9g, w9u, w9d)
    y34 = _combine_gather(ys.reshape(NTOT * RS, DS), idx)
    out = _combine(y34.reshape(3 * N, D), wts)
    return out.reshape(b, l, d), aux.reshape(())


# speedup vs baseline: 1.4731x; 1.2864x over previous
import functools

import jax
import jax.numpy as jnp
from jax.experimental import pallas as pl
from jax.experimental.pallas import tpu as pltpu
from jax.experimental.pallas import tpu_sc as plsc

N = 2048
D = 1024
II = 2816
E = 8
T = 256
NT_EXP = 23
NROW = NT_EXP * T
NC = 2
IC = II // NC
RS = 4
DS = D // RS
SC_WIN = 128


def _router_kernel(x_ref, gw_ref, slots_ref, meta_ref, wts_ref, aux_ref):
    x = x_ref[...]
    gw = gw_ref[...]
    logits = jax.lax.dot_general(
        x, gw, (((1,), (1,)), ((), ())), preferred_element_type=jnp.float32)
    ids = jax.lax.broadcasted_iota(jnp.int32, (N, E), 1).astype(jnp.float32)
    m1 = jnp.max(logits, axis=1, keepdims=True)
    i1 = jnp.min(jnp.where(logits == m1, ids, float(E)), axis=1, keepdims=True)
    masked = jnp.where(ids == i1, -jnp.inf, logits)
    m2 = jnp.max(masked, axis=1, keepdims=True)
    i2 = jnp.min(jnp.where(masked == m2, ids, float(E)), axis=1, keepdims=True)
    e2 = jnp.exp(m2 - m1)
    w1 = 1.0 / (1.0 + e2)
    wts_ref[...] = jnp.concatenate([w1, e2 * w1], axis=1)
    p = jnp.exp(logits - m1)
    probs = p / jnp.sum(p, axis=1, keepdims=True)
    meanprob = jnp.mean(probs, axis=0, keepdims=True)
    oh1 = (ids == i1).astype(jnp.float32)
    oh2 = (ids == i2).astype(jnp.float32)
    counts = jnp.sum(oh1 + oh2, axis=0, keepdims=True)
    aux_ref[...] = (0.01 * E / N) * jnp.sum(counts * meanprob).reshape(1, 1)
    m = jnp.concatenate([oh1, oh2], axis=0)
    a = m
    s = 1
    while s < 2 * N:
        a = a + jnp.concatenate(
            [jnp.zeros((s, E), jnp.float32), a[: 2 * N - s]], axis=0)
        s *= 2
    r = jnp.sum((a - m) * m, axis=1, keepdims=True)
    tiles = jnp.floor((counts + float(T - 1)) / float(T))
    ct = tiles
    s = 1
    while s < E:
        ct = ct + jnp.concatenate(
            [jnp.zeros((1, s), jnp.float32), ct[:, : E - s]], axis=1)
        s *= 2
    base = float(T) * (ct - tiles)
    slot = jnp.sum(m * base, axis=1, keepdims=True) + r
    slots_ref[...] = slot.astype(jnp.int32)
    jvec = jax.lax.broadcasted_iota(
        jnp.int32, (NT_EXP + 1, 1), 0).astype(jnp.float32)
    done = jnp.sum((jvec >= ct).astype(jnp.float32), axis=1, keepdims=True)
    e_tile = jnp.minimum(done, float(E - 1))
    meta_ref[...] = jnp.where(
        jvec == float(NT_EXP), jnp.sum(tiles), e_tile).astype(jnp.int32)


def _router(x, gate_w):
    return pl.pallas_call(
        _router_kernel,
        out_shape=[
            jax.ShapeDtypeStruct((2 * N, 1), jnp.int32),
            jax.ShapeDtypeStruct((NT_EXP + 1, 1), jnp.int32),
            jax.ShapeDtypeStruct((N, 2), jnp.float32),
            jax.ShapeDtypeStruct((1, 1), jnp.float32),
        ],
    )(x, gate_w)


def _dispatch(x4, idx):
    mesh = plsc.VectorSubcoreMesh(core_axis_name="c", subcore_axis_name="s")
    nwin = 2 * N * RS // SC_WIN
    nblk = N * RS // SC_WIN

    @functools.partial(
        pl.kernel,
        out_type=jax.ShapeDtypeStruct((NROW * RS, DS), jnp.float32),
        mesh=mesh,
        scratch_types=[],
    )
    def k(x_hbm, i_hbm, o_hbm):
        def body(x_vmem, i_vmem):
            pltpu.sync_copy(x_vmem, o_hbm.at[i_vmem.at[0]])

        pltpu.emit_pipeline(
            body,
            grid=(nwin,),
            in_specs=[
                pl.BlockSpec((SC_WIN, DS), index_map=lambda i: (i % nblk, 0)),
                pl.BlockSpec((1, SC_WIN), index_map=lambda i: (i, 0)),
            ],
            out_specs=[],
            core_axis_name=("c", "s"),
            dimension_semantics=(pltpu.PARALLEL,),
        )(x_hbm, i_hbm)

    return k(x4, idx)


def _mlp_kernel(meta_ref, xs_ref, wg_ref, wu_ref, wd_ref, out_ref):
    c = pl.program_id(0)
    j = pl.program_id(1)
    nt_used = meta_ref[NT_EXP]

    @pl.when(j < nt_used)
    def _():
        x = xs_ref[...].astype(jnp.bfloat16)
        wg = wg_ref[0].astype(jnp.bfloat16)
        wu = wu_ref[0].astype(jnp.bfloat16)
        wd = wd_ref[0].astype(jnp.bfloat16)
        g = jax.lax.dot_general(
            x, wg, (((1,), (1,)), ((), ())), preferred_element_type=jnp.float32)
        u = jax.lax.dot_general(
            x, wu, (((1,), (1,)), ((), ())), preferred_element_type=jnp.float32)
        h = (g * jax.lax.logistic(g) * u).astype(jnp.bfloat16)
        out_ref[0] = jax.lax.dot_general(
            h, wd, (((1,), (1,)), ((), ())), preferred_element_type=jnp.float32)


def _grouped_mlp(meta, xs, wg, wu, wd):
    grid_spec = pltpu.PrefetchScalarGridSpec(
        num_scalar_prefetch=1,
        grid=(NC, NT_EXP),
        in_specs=[
            pl.BlockSpec((T, D), lambda c, j, meta: (j, 0)),
            pl.BlockSpec((1, IC, D), lambda c, j, meta: (meta[j], c, 0)),
            pl.BlockSpec((1, IC, D), lambda c, j, meta: (meta[j], c, 0)),
            pl.BlockSpec((1, D, IC), lambda c, j, meta: (meta[j], 0, c)),
        ],
        out_specs=pl.BlockSpec((1, T, D), lambda c, j, meta: (c, j, 0)),
    )
    return pl.pallas_call(
        _mlp_kernel,
        grid_spec=grid_spec,
        out_shape=jax.ShapeDtypeStruct((NC, NROW, D), jnp.float32),
    )(meta, xs, wg, wu, wd)


def _shared_kernel(x_ref, wg_ref, wu_ref, wd_ref, out_ref):
    x = x_ref[...].astype(jnp.bfloat16)
    g = jax.lax.dot_general(
        x, wg_ref[...], (((1,), (0,)), ((), ())),
        preferred_element_type=jnp.float32)
    u = jax.lax.dot_general(
        x, wu_ref[...], (((1,), (0,)), ((), ())),
        preferred_element_type=jnp.float32)
    h = (g * jax.lax.logistic(g) * u).astype(jnp.bfloat16)
    out_ref[...] = jax.lax.dot_general(
        h, wd_ref[...], (((1,), (0,)), ((), ())),
        preferred_element_type=jnp.float32)


def _shared_mlp(x, swg_t, swu_t, swd_t):
    return pl.pallas_call(
        _shared_kernel,
        grid=(N // T,),
        in_specs=[
            pl.BlockSpec((T, D), lambda i: (i, 0)),
            pl.BlockSpec((D, II), lambda i: (0, 0)),
            pl.BlockSpec((D, II), lambda i: (0, 0)),
            pl.BlockSpec((II, D), lambda i: (0, 0)),
        ],
        out_specs=pl.BlockSpec((T, D), lambda i: (i, 0)),
        out_shape=jax.ShapeDtypeStruct((N, D), jnp.float32),
    )(x, swg_t, swu_t, swd_t)


def _combine_gather(ys4, idx):
    mesh = plsc.VectorSubcoreMesh(core_axis_name="c", subcore_axis_name="s")
    nwin = NC * 2 * N * RS // SC_WIN

    @functools.partial(
        pl.kernel,
        out_type=jax.ShapeDtypeStruct((NC * 2 * N * RS, DS), jnp.float32),
        mesh=mesh,
        scratch_types=[],
    )
    def k(y_hbm, i_hbm, o_hbm):
        def body(i_vmem, o_vmem):
            pltpu.sync_copy(y_hbm.at[i_vmem.at[0]], o_vmem)

        pltpu.emit_pipeline(
            body,
            grid=(nwin,),
            in_specs=[pl.BlockSpec((1, SC_WIN), index_map=lambda i: (i, 0))],
            out_specs=[pl.BlockSpec((SC_WIN, DS), index_map=lambda i: (i, 0))],
            core_axis_name=("c", "s"),
            dimension_semantics=(pltpu.PARALLEL,),
        )(i_hbm, o_hbm)

    return k(ys4, idx)


def _combine_kernel(y00_ref, y01_ref, y10_ref, y11_ref, ysh_ref, w_ref,
                    o_ref):
    w = w_ref[...]
    o_ref[...] = ((y00_ref[...] + y10_ref[...]) * w[:, 0:1]
                  + (y01_ref[...] + y11_ref[...]) * w[:, 1:2]
                  + ysh_ref[...])


def _combine(y4, ysh, wts):
    nb = N // T
    return pl.pallas_call(
        _combine_kernel,
        grid=(nb,),
        in_specs=[
            pl.BlockSpec((T, D), lambda i: (i, 0)),
            pl.BlockSpec((T, D), lambda i: (i + nb, 0)),
            pl.BlockSpec((T, D), lambda i: (i + 2 * nb, 0)),
            pl.BlockSpec((T, D), lambda i: (i + 3 * nb, 0)),
            pl.BlockSpec((T, D), lambda i: (i, 0)),
            pl.BlockSpec((T, 2), lambda i: (i, 0)),
        ],
        out_specs=pl.BlockSpec((T, D), lambda i: (i, 0)),
        out_shape=jax.ShapeDtypeStruct((N, D), jnp.float32),
    )(y4, y4, y4, y4, ysh, wts)


def kernel(hidden_states, gate_W, Wg, Wu, Wd, sWg, sWu, sWd):
    b, l, d = hidden_states.shape
    x = hidden_states.reshape(l, d)
    swg_t = sWg.T.astype(jnp.bfloat16)
    swu_t = sWu.T.astype(jnp.bfloat16)
    swd_t = sWd.T.astype(jnp.bfloat16)
    slots, meta, wts, aux = _router(x, gate_W)
    idx = (slots.reshape(-1)[:, None] * RS
           + jnp.arange(RS, dtype=jnp.int32)[None, :]).reshape(-1, SC_WIN)
    xs4 = _dispatch(x.reshape(N * RS, DS), idx)
    ysh = _shared_mlp(x, swg_t, swu_t, swd_t)
    ys = _grouped_mlp(meta.reshape(-1), xs4.reshape(NROW, D), Wg, Wu, Wd)
    idxf = idx.reshape(-1)
    idx2 = jnp.concatenate([idxf, idxf + NROW * RS]).reshape(-1, SC_WIN)
    y4 = _combine_gather(ys.reshape(NC * NROW * RS, DS), idx2)
    out = _combine(y4.reshape(NC * 2 * N, D), ysh, wts)
    return out.reshape(b, l, d), aux.reshape(())


# speedup vs baseline: 1.6013x; 1.0871x over previous
import functools

import jax
import jax.numpy as jnp
from jax.experimental import pallas as pl
from jax.experimental.pallas import tpu as pltpu
from jax.experimental.pallas import tpu_sc as plsc

N = 2048
D = 1024
II = 2816
E = 8
T = 256
NT_EXP = 23
NROW = NT_EXP * T
NC = 2
IC = II // NC
RS = 4
DS = D // RS
SC_WIN = 128


def _router_kernel(x_ref, gw_ref, slots_ref, meta_ref, wts_ref, aux_ref):
    x = x_ref[...]
    gw = gw_ref[...]
    logits = jax.lax.dot_general(
        x, gw, (((1,), (1,)), ((), ())), preferred_element_type=jnp.float32)
    ids = jax.lax.broadcasted_iota(jnp.int32, (N, E), 1).astype(jnp.float32)
    m1 = jnp.max(logits, axis=1, keepdims=True)
    i1 = jnp.min(jnp.where(logits == m1, ids, float(E)), axis=1, keepdims=True)
    masked = jnp.where(ids == i1, -jnp.inf, logits)
    m2 = jnp.max(masked, axis=1, keepdims=True)
    i2 = jnp.min(jnp.where(masked == m2, ids, float(E)), axis=1, keepdims=True)
    e2 = jnp.exp(m2 - m1)
    w1 = 1.0 / (1.0 + e2)
    wts_ref[...] = jnp.concatenate([w1, e2 * w1], axis=1)
    p = jnp.exp(logits - m1)
    probs = p / jnp.sum(p, axis=1, keepdims=True)
    meanprob = jnp.mean(probs, axis=0, keepdims=True)
    oh1 = (ids == i1).astype(jnp.float32)
    oh2 = (ids == i2).astype(jnp.float32)
    counts = jnp.sum(oh1 + oh2, axis=0, keepdims=True)
    aux_ref[...] = (0.01 * E / N) * jnp.sum(counts * meanprob).reshape(1, 1)
    m = jnp.concatenate([oh1, oh2], axis=0)
    a = m
    s = 1
    while s < 2 * N:
        a = a + jnp.concatenate(
            [jnp.zeros((s, E), jnp.float32), a[: 2 * N - s]], axis=0)
        s *= 2
    r = jnp.sum((a - m) * m, axis=1, keepdims=True)
    tiles = jnp.floor((counts + float(T - 1)) / float(T))
    ct = tiles
    s = 1
    while s < E:
        ct = ct + jnp.concatenate(
            [jnp.zeros((1, s), jnp.float32), ct[:, : E - s]], axis=1)
        s *= 2
    base = float(T) * (ct - tiles)
    slot = jnp.sum(m * base, axis=1, keepdims=True) + r
    slots_ref[...] = slot.astype(jnp.int32)
    jvec = jax.lax.broadcasted_iota(
        jnp.int32, (NT_EXP + 1, 1), 0).astype(jnp.float32)
    done = jnp.sum((jvec >= ct).astype(jnp.float32), axis=1, keepdims=True)
    e_tile = jnp.minimum(done, float(E - 1))
    meta_ref[...] = jnp.where(
        jvec == float(NT_EXP), jnp.sum(tiles), e_tile).astype(jnp.int32)


def _router(x, gate_w):
    return pl.pallas_call(
        _router_kernel,
        out_shape=[
            jax.ShapeDtypeStruct((2 * N, 1), jnp.int32),
            jax.ShapeDtypeStruct((NT_EXP + 1, 1), jnp.int32),
            jax.ShapeDtypeStruct((N, 2), jnp.float32),
            jax.ShapeDtypeStruct((1, 1), jnp.float32),
        ],
    )(x, gate_w)


def _dispatch(x4, idx):
    mesh = plsc.VectorSubcoreMesh(core_axis_name="c", subcore_axis_name="s")
    nwin = 2 * N * RS // SC_WIN
    nblk = N * RS // SC_WIN

    @functools.partial(
        pl.kernel,
        out_type=jax.ShapeDtypeStruct((NROW * RS, DS), jnp.float32),
        mesh=mesh,
        scratch_types=[],
    )
    def k(x_hbm, i_hbm, o_hbm):
        def body(x_vmem, i_vmem):
            pltpu.sync_copy(x_vmem, o_hbm.at[i_vmem.at[0]])

        pltpu.emit_pipeline(
            body,
            grid=(nwin,),
            in_specs=[
                pl.BlockSpec((SC_WIN, DS), index_map=lambda i: (i % nblk, 0)),
                pl.BlockSpec((1, SC_WIN), index_map=lambda i: (i, 0)),
            ],
            out_specs=[],
            core_axis_name=("c", "s"),
            dimension_semantics=(pltpu.PARALLEL,),
        )(x_hbm, i_hbm)

    return k(x4, idx)


def _mlp_kernel(meta_ref, xs_ref, wg_ref, wu_ref, wd_ref, acc_ref, out_ref):
    c = pl.program_id(0)
    j = pl.program_id(1)
    nt_used = meta_ref[NT_EXP]

    @pl.when(j < nt_used)
    def _():
        x = xs_ref[...].astype(jnp.bfloat16)
        wg = wg_ref[0].astype(jnp.bfloat16)
        wu = wu_ref[0].astype(jnp.bfloat16)
        wd = wd_ref[0].astype(jnp.bfloat16)
        g = jax.lax.dot_general(
            x, wg, (((1,), (1,)), ((), ())), preferred_element_type=jnp.float32)
        u = jax.lax.dot_general(
            x, wu, (((1,), (1,)), ((), ())), preferred_element_type=jnp.float32)
        h = (g * jax.lax.logistic(g) * u).astype(jnp.bfloat16)
        part = jax.lax.dot_general(
            h, wd, (((1,), (1,)), ((), ())), preferred_element_type=jnp.float32)

        @pl.when(c == 0)
        def _():
            out_ref[...] = part

        @pl.when(c > 0)
        def _():
            out_ref[...] = acc_ref[...] + part


def _grouped_mlp(meta, xs, wg, wu, wd, acc):
    grid_spec = pltpu.PrefetchScalarGridSpec(
        num_scalar_prefetch=1,
        grid=(NC, NT_EXP),
        in_specs=[
            pl.BlockSpec((T, D), lambda c, j, meta: (j, 0)),
            pl.BlockSpec((1, IC, D), lambda c, j, meta: (meta[j], c, 0)),
            pl.BlockSpec((1, IC, D), lambda c, j, meta: (meta[j], c, 0)),
            pl.BlockSpec((1, D, IC), lambda c, j, meta: (meta[j], 0, c)),
            pl.BlockSpec((T, D), lambda c, j, meta: (j, 0)),
        ],
        out_specs=pl.BlockSpec((T, D), lambda c, j, meta: (j, 0)),
    )
    return pl.pallas_call(
        _mlp_kernel,
        grid_spec=grid_spec,
        out_shape=jax.ShapeDtypeStruct((NROW, D), jnp.float32),
        input_output_aliases={5: 0},
    )(meta, xs, wg, wu, wd, acc)


def _shared_kernel(x_ref, wg_ref, wu_ref, wd_ref, out_ref):
    x = x_ref[...].astype(jnp.bfloat16)
    g = jax.lax.dot_general(
        x, wg_ref[...], (((1,), (0,)), ((), ())),
        preferred_element_type=jnp.float32)
    u = jax.lax.dot_general(
        x, wu_ref[...], (((1,), (0,)), ((), ())),
        preferred_element_type=jnp.float32)
    h = (g * jax.lax.logistic(g) * u).astype(jnp.bfloat16)
    out_ref[...] = jax.lax.dot_general(
        h, wd_ref[...], (((1,), (0,)), ((), ())),
        preferred_element_type=jnp.float32)


def _shared_mlp(x, swg_t, swu_t, swd_t):
    return pl.pallas_call(
        _shared_kernel,
        grid=(N // T,),
        in_specs=[
            pl.BlockSpec((T, D), lambda i: (i, 0)),
            pl.BlockSpec((D, II), lambda i: (0, 0)),
            pl.BlockSpec((D, II), lambda i: (0, 0)),
            pl.BlockSpec((II, D), lambda i: (0, 0)),
        ],
        out_specs=pl.BlockSpec((T, D), lambda i: (i, 0)),
        out_shape=jax.ShapeDtypeStruct((N, D), jnp.float32),
    )(x, swg_t, swu_t, swd_t)


def _combine_gather(ys4, idx):
    mesh = plsc.VectorSubcoreMesh(core_axis_name="c", subcore_axis_name="s")
    nwin = 2 * N * RS // SC_WIN

    @functools.partial(
        pl.kernel,
        out_type=jax.ShapeDtypeStruct((2 * N * RS, DS), jnp.float32),
        mesh=mesh,
        scratch_types=[],
    )
    def k(y_hbm, i_hbm, o_hbm):
        def body(i_vmem, o_vmem):
            pltpu.sync_copy(y_hbm.at[i_vmem.at[0]], o_vmem)

        pltpu.emit_pipeline(
            body,
            grid=(nwin,),
            in_specs=[pl.BlockSpec((1, SC_WIN), index_map=lambda i: (i, 0))],
            out_specs=[pl.BlockSpec((SC_WIN, DS), index_map=lambda i: (i, 0))],
            core_axis_name=("c", "s"),
            dimension_semantics=(pltpu.PARALLEL,),
        )(i_hbm, o_hbm)

    return k(ys4, idx)


def _combine_kernel(y0_ref, y1_ref, ysh_ref, w_ref, o_ref):
    w = w_ref[...]
    o_ref[...] = (y0_ref[...] * w[:, 0:1] + y1_ref[...] * w[:, 1:2]
                  + ysh_ref[...])


def _combine(y2, ysh, wts):
    nb = N // T
    return pl.pallas_call(
        _combine_kernel,
        grid=(nb,),
        in_specs=[
            pl.BlockSpec((T, D), lambda i: (i, 0)),
            pl.BlockSpec((T, D), lambda i: (i + nb, 0)),
            pl.BlockSpec((T, D), lambda i: (i, 0)),
            pl.BlockSpec((T, 2), lambda i: (i, 0)),
        ],
        out_specs=pl.BlockSpec((T, D), lambda i: (i, 0)),
        out_shape=jax.ShapeDtypeStruct((N, D), jnp.float32),
    )(y2, y2, ysh, wts)


def kernel(hidden_states, gate_W, Wg, Wu, Wd, sWg, sWu, sWd):
    b, l, d = hidden_states.shape
    x = hidden_states.reshape(l, d)
    swg_t = sWg.T.astype(jnp.bfloat16)
    swu_t = sWu.T.astype(jnp.bfloat16)
    swd_t = sWd.T.astype(jnp.bfloat16)
    slots, meta, wts, aux = _router(x, gate_W)
    idx = (slots.reshape(-1)[:, None] * RS
           + jnp.arange(RS, dtype=jnp.int32)[None, :]).reshape(-1, SC_WIN)
    xs4 = _dispatch(x.reshape(N * RS, DS), idx)
    ysh = _shared_mlp(x, swg_t, swu_t, swd_t)
    acc = jnp.zeros((NROW, D), jnp.float32)
    ys = _grouped_mlp(meta.reshape(-1), xs4.reshape(NROW, D), Wg, Wu, Wd, acc)
    y2 = _combine_gather(ys.reshape(NROW * RS, DS), idx)
    out = _combine(y2.reshape(2 * N, D), ysh, wts)
    return out.reshape(b, l, d), aux.reshape(())


# speedup vs baseline: 1.6082x; 1.0043x over previous
import functools

import jax
import jax.numpy as jnp
from jax.experimental import pallas as pl
from jax.experimental.pallas import tpu as pltpu
from jax.experimental.pallas import tpu_sc as plsc

N = 2048
D = 1024
II = 2816
E = 8
T = 256
NT_EXP = 23
NROW = NT_EXP * T
NC = 2
IC = II // NC
RS = 4
DS = D // RS
SC_WIN = 128


def _router_kernel(x_ref, gw_ref, slots_ref, meta_ref, wts_ref, aux_ref):
    x = x_ref[...]
    gw = gw_ref[...]
    logits = jax.lax.dot_general(
        x, gw, (((1,), (1,)), ((), ())), preferred_element_type=jnp.float32)
    ids = jax.lax.broadcasted_iota(jnp.int32, (N, E), 1).astype(jnp.float32)
    m1 = jnp.max(logits, axis=1, keepdims=True)
    i1 = jnp.min(jnp.where(logits == m1, ids, float(E)), axis=1, keepdims=True)
    masked = jnp.where(ids == i1, -jnp.inf, logits)
    m2 = jnp.max(masked, axis=1, keepdims=True)
    i2 = jnp.min(jnp.where(masked == m2, ids, float(E)), axis=1, keepdims=True)
    e2 = jnp.exp(m2 - m1)
    w1 = 1.0 / (1.0 + e2)
    wts_ref[...] = jnp.concatenate([w1, e2 * w1], axis=1)
    p = jnp.exp(logits - m1)
    probs = p / jnp.sum(p, axis=1, keepdims=True)
    meanprob = jnp.mean(probs, axis=0, keepdims=True)
    oh1 = (ids == i1).astype(jnp.float32)
    oh2 = (ids == i2).astype(jnp.float32)
    counts = jnp.sum(oh1 + oh2, axis=0, keepdims=True)
    aux_ref[...] = (0.01 * E / N) * jnp.sum(counts * meanprob).reshape(1, 1)
    m = jnp.concatenate([oh1, oh2], axis=0)
    a = m
    s = 1
    while s < 2 * N:
        a = a + jnp.concatenate(
            [jnp.zeros((s, E), jnp.float32), a[: 2 * N - s]], axis=0)
        s *= 2
    r = jnp.sum((a - m) * m, axis=1, keepdims=True)
    tiles = jnp.floor((counts + float(T - 1)) / float(T))
    ct = tiles
    s = 1
    while s < E:
        ct = ct + jnp.concatenate(
            [jnp.zeros((1, s), jnp.float32), ct[:, : E - s]], axis=1)
        s *= 2
    base = float(T) * (ct - tiles)
    slot = jnp.sum(m * base, axis=1, keepdims=True) + r
    slots_ref[...] = slot.astype(jnp.int32)
    jvec = jax.lax.broadcasted_iota(
        jnp.int32, (NT_EXP + 1, 1), 0).astype(jnp.float32)
    done = jnp.sum((jvec >= ct).astype(jnp.float32), axis=1, keepdims=True)
    e_tile = jnp.minimum(done, float(E - 1))
    meta_ref[...] = jnp.where(
        jvec == float(NT_EXP), jnp.sum(tiles), e_tile).astype(jnp.int32)


def _router(x, gate_w):
    return pl.pallas_call(
        _router_kernel,
        out_shape=[
            jax.ShapeDtypeStruct((2 * N, 1), jnp.int32),
            jax.ShapeDtypeStruct((NT_EXP + 1, 1), jnp.int32),
            jax.ShapeDtypeStruct((N, 2), jnp.float32),
            jax.ShapeDtypeStruct((1, 1), jnp.float32),
        ],
    )(x, gate_w)


def _dispatch(x4, idx):
    mesh = plsc.VectorSubcoreMesh(core_axis_name="c", subcore_axis_name="s")
    nwin = 2 * N * RS // SC_WIN
    nblk = N * RS // SC_WIN

    @functools.partial(
        pl.kernel,
        out_type=jax.ShapeDtypeStruct((NROW * RS, DS), jnp.float32),
        mesh=mesh,
        scratch_types=[],
    )
    def k(x_hbm, i_hbm, o_hbm):
        def body(x_vmem, i_vmem):
            pltpu.sync_copy(x_vmem, o_hbm.at[i_vmem.at[0]])

        pltpu.emit_pipeline(
            body,
            grid=(nwin,),
            in_specs=[
                pl.BlockSpec((SC_WIN, DS), index_map=lambda i: (i % nblk, 0)),
                pl.BlockSpec((1, SC_WIN), index_map=lambda i: (i, 0)),
            ],
            out_specs=[],
            core_axis_name=("c", "s"),
            dimension_semantics=(pltpu.PARALLEL,),
        )(x_hbm, i_hbm)

    return k(x4, idx)


def _mlp_kernel(meta_ref, xs_ref, wg_ref, wu_ref, wd_ref, acc_ref, out_ref):
    c = pl.program_id(0)
    j = pl.program_id(1)
    nt_used = meta_ref[NT_EXP]

    @pl.when(j < nt_used)
    def _():
        x = xs_ref[...].astype(jnp.bfloat16)
        wg = wg_ref[0].astype(jnp.bfloat16)
        wu = wu_ref[0].astype(jnp.bfloat16)
        wd = wd_ref[0].astype(jnp.bfloat16)
        g = jax.lax.dot_general(
            x, wg, (((1,), (1,)), ((), ())), preferred_element_type=jnp.float32)
        u = jax.lax.dot_general(
            x, wu, (((1,), (1,)), ((), ())), preferred_element_type=jnp.float32)
        h = (g * jax.lax.logistic(g) * u).astype(jnp.bfloat16)
        part = jax.lax.dot_general(
            h, wd, (((1,), (1,)), ((), ())), preferred_element_type=jnp.float32)

        @pl.when(c == 0)
        def _():
            out_ref[...] = part

        @pl.when(c > 0)
        def _():
            out_ref[...] = acc_ref[...] + part


def _grouped_mlp(meta, xs, wg, wu, wd, acc):
    grid_spec = pltpu.PrefetchScalarGridSpec(
        num_scalar_prefetch=1,
        grid=(NC, NT_EXP),
        in_specs=[
            pl.BlockSpec((T, D), lambda c, j, meta: (j, 0)),
            pl.BlockSpec((1, IC, D), lambda c, j, meta: (meta[j], c, 0)),
            pl.BlockSpec((1, IC, D), lambda c, j, meta: (meta[j], c, 0)),
            pl.BlockSpec((1, D, IC), lambda c, j, meta: (meta[j], 0, c)),
            pl.BlockSpec((T, D), lambda c, j, meta: (j, 0)),
        ],
        out_specs=pl.BlockSpec((T, D), lambda c, j, meta: (j, 0)),
    )
    return pl.pallas_call(
        _mlp_kernel,
        grid_spec=grid_spec,
        out_shape=jax.ShapeDtypeStruct((NROW, D), jnp.float32),
        input_output_aliases={5: 0},
    )(meta, xs, wg, wu, wd, acc)


def _shared_kernel(x_ref, wg_ref, wu_ref, wd_ref, out_ref):
    x = x_ref[...].astype(jnp.bfloat16)
    g = jax.lax.dot_general(
        x, wg_ref[...], (((1,), (1,)), ((), ())),
        preferred_element_type=jnp.float32)
    u = jax.lax.dot_general(
        x, wu_ref[...], (((1,), (1,)), ((), ())),
        preferred_element_type=jnp.float32)
    h = (g * jax.lax.logistic(g) * u).astype(jnp.bfloat16)
    out_ref[...] = jax.lax.dot_general(
        h, wd_ref[...], (((1,), (1,)), ((), ())),
        preferred_element_type=jnp.float32)


def _shared_mlp(x, swg, swu, swd):
    return pl.pallas_call(
        _shared_kernel,
        grid=(N // T,),
        in_specs=[
            pl.BlockSpec((T, D), lambda i: (i, 0)),
            pl.BlockSpec((II, D), lambda i: (0, 0)),
            pl.BlockSpec((II, D), lambda i: (0, 0)),
            pl.BlockSpec((D, II), lambda i: (0, 0)),
        ],
        out_specs=pl.BlockSpec((T, D), lambda i: (i, 0)),
        out_shape=jax.ShapeDtypeStruct((N, D), jnp.float32),
    )(x, swg, swu, swd)


def _combine_gather(ys4, idx):
    mesh = plsc.VectorSubcoreMesh(core_axis_name="c", subcore_axis_name="s")
    nwin = 2 * N * RS // SC_WIN

    @functools.partial(
        pl.kernel,
        out_type=jax.ShapeDtypeStruct((2 * N * RS, DS), jnp.float32),
        mesh=mesh,
        scratch_types=[],
    )
    def k(y_hbm, i_hbm, o_hbm):
        def body(i_vmem, o_vmem):
            pltpu.sync_copy(y_hbm.at[i_vmem.at[0]], o_vmem)

        pltpu.emit_pipeline(
            body,
            grid=(nwin,),
            in_specs=[pl.BlockSpec((1, SC_WIN), index_map=lambda i: (i, 0))],
            out_specs=[pl.BlockSpec((SC_WIN, DS), index_map=lambda i: (i, 0))],
            core_axis_name=("c", "s"),
            dimension_semantics=(pltpu.PARALLEL,),
        )(i_hbm, o_hbm)

    return k(ys4, idx)


def _combine_kernel(y0_ref, y1_ref, ysh_ref, w_ref, o_ref):
    w = w_ref[...]
    o_ref[...] = (y0_ref[...] * w[:, 0:1] + y1_ref[...] * w[:, 1:2]
                  + ysh_ref[...])


def _combine(y2, ysh, wts):
    nb = N // T
    return pl.pallas_call(
        _combine_kernel,
        grid=(nb,),
        in_specs=[
            pl.BlockSpec((T, D), lambda i: (i, 0)),
            pl.BlockSpec((T, D), lambda i: (i + nb, 0)),
            pl.BlockSpec((T, D), lambda i: (i, 0)),
            pl.BlockSpec((T, 2), lambda i: (i, 0)),
        ],
        out_specs=pl.BlockSpec((T, D), lambda i: (i, 0)),
        out_shape=jax.ShapeDtypeStruct((N, D), jnp.float32),
    )(y2, y2, ysh, wts)


def kernel(hidden_states, gate_W, Wg, Wu, Wd, sWg, sWu, sWd):
    b, l, d = hidden_states.shape
    x = hidden_states.reshape(l, d)
    swg_t = sWg.astype(jnp.bfloat16)
    swu_t = sWu.astype(jnp.bfloat16)
    swd_t = sWd.astype(jnp.bfloat16)
    slots, meta, wts, aux = _router(x, gate_W)
    idx = (slots.reshape(-1)[:, None] * RS
           + jnp.arange(RS, dtype=jnp.int32)[None, :]).reshape(-1, SC_WIN)
    xs4 = _dispatch(x.reshape(N * RS, DS), idx)
    ysh = _shared_mlp(x, swg_t, swu_t, swd_t)
    acc = jnp.zeros((NROW, D), jnp.float32)
    ys = _grouped_mlp(meta.reshape(-1), xs4.reshape(NROW, D), Wg, Wu, Wd, acc)
    y2 = _combine_gather(ys.reshape(NROW * RS, DS), idx)
    out = _combine(y2.reshape(2 * N, D), ysh, wts)
    return out.reshape(b, l, d), aux.reshape(())


# speedup vs baseline: 1.6327x; 1.0152x over previous
import functools

import jax
import jax.numpy as jnp
from jax.experimental import pallas as pl
from jax.experimental.pallas import tpu as pltpu
from jax.experimental.pallas import tpu_sc as plsc

N = 2048
D = 1024
II = 2816
E = 8
T = 256
TM = 512
NT_EXP = 2 * N // TM + E - 1
NROW = NT_EXP * TM
NC = 2
IC = II // NC
RS = 4
DS = D // RS
SC_WIN = 128


def _router_kernel(x_ref, gw_ref, slots_ref, meta_ref, wts_ref, aux_ref):
    x = x_ref[...]
    gw = gw_ref[...]
    logits = jax.lax.dot_general(
        x, gw, (((1,), (1,)), ((), ())), preferred_element_type=jnp.float32)
    ids = jax.lax.broadcasted_iota(jnp.int32, (N, E), 1).astype(jnp.float32)
    m1 = jnp.max(logits, axis=1, keepdims=True)
    i1 = jnp.min(jnp.where(logits == m1, ids, float(E)), axis=1, keepdims=True)
    masked = jnp.where(ids == i1, -jnp.inf, logits)
    m2 = jnp.max(masked, axis=1, keepdims=True)
    i2 = jnp.min(jnp.where(masked == m2, ids, float(E)), axis=1, keepdims=True)
    e2 = jnp.exp(m2 - m1)
    w1 = 1.0 / (1.0 + e2)
    wts_ref[...] = jnp.concatenate([w1, e2 * w1], axis=1)
    p = jnp.exp(logits - m1)
    probs = p / jnp.sum(p, axis=1, keepdims=True)
    meanprob = jnp.mean(probs, axis=0, keepdims=True)
    oh1 = (ids == i1).astype(jnp.float32)
    oh2 = (ids == i2).astype(jnp.float32)
    counts = jnp.sum(oh1 + oh2, axis=0, keepdims=True)
    aux_ref[...] = (0.01 * E / N) * jnp.sum(counts * meanprob).reshape(1, 1)
    m = jnp.concatenate([oh1, oh2], axis=0)
    a = m
    s = 1
    while s < 2 * N:
        a = a + jnp.concatenate(
            [jnp.zeros((s, E), jnp.float32), a[: 2 * N - s]], axis=0)
        s *= 2
    r = jnp.sum((a - m) * m, axis=1, keepdims=True)
    tiles = jnp.floor((counts + float(TM - 1)) / float(TM))
    ct = tiles
    s = 1
    while s < E:
        ct = ct + jnp.concatenate(
            [jnp.zeros((1, s), jnp.float32), ct[:, : E - s]], axis=1)
        s *= 2
    base = float(TM) * (ct - tiles)
    slot = jnp.sum(m * base, axis=1, keepdims=True) + r
    slots_ref[...] = slot.astype(jnp.int32)
    jvec = jax.lax.broadcasted_iota(
        jnp.int32, (NT_EXP + 1, 1), 0).astype(jnp.float32)
    done = jnp.sum((jvec >= ct).astype(jnp.float32), axis=1, keepdims=True)
    e_tile = jnp.minimum(done, float(E - 1))
    meta_ref[...] = jnp.where(
        jvec == float(NT_EXP), jnp.sum(tiles), e_tile).astype(jnp.int32)


def _router(x, gate_w):
    return pl.pallas_call(
        _router_kernel,
        out_shape=[
            jax.ShapeDtypeStruct((2 * N, 1), jnp.int32),
            jax.ShapeDtypeStruct((NT_EXP + 1, 1), jnp.int32),
            jax.ShapeDtypeStruct((N, 2), jnp.float32),
            jax.ShapeDtypeStruct((1, 1), jnp.float32),
        ],
    )(x, gate_w)


def _dispatch(x4, idx):
    mesh = plsc.VectorSubcoreMesh(core_axis_name="c", subcore_axis_name="s")
    nwin = 2 * N * RS // SC_WIN
    nblk = N * RS // SC_WIN

    @functools.partial(
        pl.kernel,
        out_type=jax.ShapeDtypeStruct((NROW * RS, DS), jnp.float32),
        mesh=mesh,
        scratch_types=[],
    )
    def k(x_hbm, i_hbm, o_hbm):
        def body(x_vmem, i_vmem):
            pltpu.sync_copy(x_vmem, o_hbm.at[i_vmem.at[0]])

        pltpu.emit_pipeline(
            body,
            grid=(nwin,),
            in_specs=[
                pl.BlockSpec((SC_WIN, DS), index_map=lambda i: (i % nblk, 0)),
                pl.BlockSpec((1, SC_WIN), index_map=lambda i: (i, 0)),
            ],
            out_specs=[],
            core_axis_name=("c", "s"),
            dimension_semantics=(pltpu.PARALLEL,),
        )(x_hbm, i_hbm)

    return k(x4, idx)


def _mlp_kernel(meta_ref, xs_ref, wg_ref, wu_ref, wd_ref, acc_ref, out_ref):
    c = pl.program_id(0)
    j = pl.program_id(1)
    nt_used = meta_ref[NT_EXP]

    @pl.when(j < nt_used)
    def _():
        x = xs_ref[...].astype(jnp.bfloat16)
        wg = wg_ref[0].astype(jnp.bfloat16)
        wu = wu_ref[0].astype(jnp.bfloat16)
        wd = wd_ref[0].astype(jnp.bfloat16)
        g = jax.lax.dot_general(
            x, wg, (((1,), (1,)), ((), ())), preferred_element_type=jnp.float32)
        u = jax.lax.dot_general(
            x, wu, (((1,), (1,)), ((), ())), preferred_element_type=jnp.float32)
        h = (g * jax.lax.logistic(g) * u).astype(jnp.bfloat16)
        part = jax.lax.dot_general(
            h, wd, (((1,), (1,)), ((), ())), preferred_element_type=jnp.float32)

        @pl.when(c == 0)
        def _():
            out_ref[...] = part

        @pl.when(c > 0)
        def _():
            out_ref[...] = acc_ref[...] + part


def _grouped_mlp(meta, xs, wg, wu, wd, acc):
    grid_spec = pltpu.PrefetchScalarGridSpec(
        num_scalar_prefetch=1,
        grid=(NC, NT_EXP),
        in_specs=[
            pl.BlockSpec((TM, D), lambda c, j, meta: (j, 0)),
            pl.BlockSpec((1, IC, D), lambda c, j, meta: (meta[j], c, 0)),
            pl.BlockSpec((1, IC, D), lambda c, j, meta: (meta[j], c, 0)),
            pl.BlockSpec((1, D, IC), lambda c, j, meta: (meta[j], 0, c)),
            pl.BlockSpec((TM, D), lambda c, j, meta: (j, 0)),
        ],
        out_specs=pl.BlockSpec((TM, D), lambda c, j, meta: (j, 0)),
    )
    return pl.pallas_call(
        _mlp_kernel,
        grid_spec=grid_spec,
        out_shape=jax.ShapeDtypeStruct((NROW, D), jnp.float32),
        input_output_aliases={5: 0},
    )(meta, xs, wg, wu, wd, acc)


def _shared_kernel(x_ref, wg_ref, wu_ref, wd_ref, out_ref):
    x = x_ref[...].astype(jnp.bfloat16)
    g = jax.lax.dot_general(
        x, wg_ref[...], (((1,), (1,)), ((), ())),
        preferred_element_type=jnp.float32)
    u = jax.lax.dot_general(
        x, wu_ref[...], (((1,), (1,)), ((), ())),
        preferred_element_type=jnp.float32)
    h = (g * jax.lax.logistic(g) * u).astype(jnp.bfloat16)
    out_ref[...] = jax.lax.dot_general(
        h, wd_ref[...], (((1,), (1,)), ((), ())),
        preferred_element_type=jnp.float32)


def _shared_mlp(x, swg, swu, swd):
    return pl.pallas_call(
        _shared_kernel,
        grid=(N // T,),
        in_specs=[
            pl.BlockSpec((T, D), lambda i: (i, 0)),
            pl.BlockSpec((II, D), lambda i: (0, 0)),
            pl.BlockSpec((II, D), lambda i: (0, 0)),
            pl.BlockSpec((D, II), lambda i: (0, 0)),
        ],
        out_specs=pl.BlockSpec((T, D), lambda i: (i, 0)),
        out_shape=jax.ShapeDtypeStruct((N, D), jnp.float32),
    )(x, swg, swu, swd)


def _combine_gather(ys4, idx):
    mesh = plsc.VectorSubcoreMesh(core_axis_name="c", subcore_axis_name="s")
    nwin = 2 * N * RS // SC_WIN

    @functools.partial(
        pl.kernel,
        out_type=jax.ShapeDtypeStruct((2 * N * RS, DS), jnp.float32),
        mesh=mesh,
        scratch_types=[],
    )
    def k(y_hbm, i_hbm, o_hbm):
        def body(i_vmem, o_vmem):
            pltpu.sync_copy(y_hbm.at[i_vmem.at[0]], o_vmem)

        pltpu.emit_pipeline(
            body,
            grid=(nwin,),
            in_specs=[pl.BlockSpec((1, SC_WIN), index_map=lambda i: (i, 0))],
            out_specs=[pl.BlockSpec((SC_WIN, DS), index_map=lambda i: (i, 0))],
            core_axis_name=("c", "s"),
            dimension_semantics=(pltpu.PARALLEL,),
        )(i_hbm, o_hbm)

    return k(ys4, idx)


def _combine_kernel(y0_ref, y1_ref, ysh_ref, w_ref, o_ref):
    w = w_ref[...]
    o_ref[...] = (y0_ref[...] * w[:, 0:1] + y1_ref[...] * w[:, 1:2]
                  + ysh_ref[...])


def _combine(y2, ysh, wts):
    nb = N // T
    return pl.pallas_call(
        _combine_kernel,
        grid=(nb,),
        in_specs=[
            pl.BlockSpec((T, D), lambda i: (i, 0)),
            pl.BlockSpec((T, D), lambda i: (i + nb, 0)),
            pl.BlockSpec((T, D), lambda i: (i, 0)),
            pl.BlockSpec((T, 2), lambda i: (i, 0)),
        ],
        out_specs=pl.BlockSpec((T, D), lambda i: (i, 0)),
        out_shape=jax.ShapeDtypeStruct((N, D), jnp.float32),
    )(y2, y2, ysh, wts)


def kernel(hidden_states, gate_W, Wg, Wu, Wd, sWg, sWu, sWd):
    b, l, d = hidden_states.shape
    x = hidden_states.reshape(l, d)
    swg_t = sWg.astype(jnp.bfloat16)
    swu_t = sWu.astype(jnp.bfloat16)
    swd_t = sWd.astype(jnp.bfloat16)
    slots, meta, wts, aux = _router(x, gate_W)
    idx = (slots.reshape(-1)[:, None] * RS
           + jnp.arange(RS, dtype=jnp.int32)[None, :]).reshape(-1, SC_WIN)
    xs4 = _dispatch(x.reshape(N * RS, DS), idx)
    ysh = _shared_mlp(x, swg_t, swu_t, swd_t)
    acc = jnp.zeros((NROW, D), jnp.float32)
    ys = _grouped_mlp(meta.reshape(-1), xs4.reshape(NROW, D), Wg, Wu, Wd, acc)
    y2 = _combine_gather(ys.reshape(NROW * RS, DS), idx)
    out = _combine(y2.reshape(2 * N, D), ysh, wts)
    return out.reshape(b, l, d), aux.reshape(())


# speedup vs baseline: 1.6405x; 1.0048x over previous
import functools

import jax
import jax.numpy as jnp
from jax.experimental import pallas as pl
from jax.experimental.pallas import tpu as pltpu
from jax.experimental.pallas import tpu_sc as plsc

N = 2048
D = 1024
II = 2816
E = 8
T = 256
TM = 512
NT_EXP = 2 * N // TM + E - 1
NROW = NT_EXP * TM
NC = 2
IC = II // NC
RS = 4
DS = D // RS
SC_WIN = 128


def _router_kernel(x_ref, gw_ref, slots_ref, meta_ref, wts_ref, aux_ref):
    x = x_ref[...]
    gw = gw_ref[...]
    logits = jax.lax.dot_general(
        x, gw, (((1,), (1,)), ((), ())), preferred_element_type=jnp.float32)
    ids = jax.lax.broadcasted_iota(jnp.int32, (N, E), 1).astype(jnp.float32)
    m1 = jnp.max(logits, axis=1, keepdims=True)
    i1 = jnp.min(jnp.where(logits == m1, ids, float(E)), axis=1, keepdims=True)
    masked = jnp.where(ids == i1, -jnp.inf, logits)
    m2 = jnp.max(masked, axis=1, keepdims=True)
    i2 = jnp.min(jnp.where(masked == m2, ids, float(E)), axis=1, keepdims=True)
    e2 = jnp.exp(m2 - m1)
    w1 = 1.0 / (1.0 + e2)
    wts_ref[...] = jnp.concatenate([w1, e2 * w1], axis=1)
    p = jnp.exp(logits - m1)
    probs = p / jnp.sum(p, axis=1, keepdims=True)
    meanprob = jnp.mean(probs, axis=0, keepdims=True)
    oh1 = (ids == i1).astype(jnp.float32)
    oh2 = (ids == i2).astype(jnp.float32)
    counts = jnp.sum(oh1 + oh2, axis=0, keepdims=True)
    aux_ref[...] = (0.01 * E / N) * jnp.sum(counts * meanprob).reshape(1, 1)
    m = jnp.concatenate([oh1, oh2], axis=0)
    a = m
    s = 1
    while s < 2 * N:
        a = a + jnp.concatenate(
            [jnp.zeros((s, E), jnp.float32), a[: 2 * N - s]], axis=0)
        s *= 2
    r = jnp.sum((a - m) * m, axis=1, keepdims=True)
    tiles = jnp.floor((counts + float(TM - 1)) / float(TM))
    ct = tiles
    s = 1
    while s < E:
        ct = ct + jnp.concatenate(
            [jnp.zeros((1, s), jnp.float32), ct[:, : E - s]], axis=1)
        s *= 2
    base = float(TM) * (ct - tiles)
    slot = jnp.sum(m * base, axis=1, keepdims=True) + r
    slots_ref[...] = slot.astype(jnp.int32)
    jvec = jax.lax.broadcasted_iota(
        jnp.int32, (NT_EXP + 1, 1), 0).astype(jnp.float32)
    done = jnp.sum((jvec >= ct).astype(jnp.float32), axis=1, keepdims=True)
    e_tile = jnp.minimum(done, float(E - 1))
    meta_ref[...] = jnp.where(
        jvec == float(NT_EXP), jnp.sum(tiles), e_tile).astype(jnp.int32)


def _router(x, gate_w):
    return pl.pallas_call(
        _router_kernel,
        out_shape=[
            jax.ShapeDtypeStruct((2 * N, 1), jnp.int32),
            jax.ShapeDtypeStruct((NT_EXP + 1, 1), jnp.int32),
            jax.ShapeDtypeStruct((N, 2), jnp.float32),
            jax.ShapeDtypeStruct((1, 1), jnp.float32),
        ],
    )(x, gate_w)


def _dispatch(x4, idx):
    mesh = plsc.VectorSubcoreMesh(core_axis_name="c", subcore_axis_name="s")
    nwin = 2 * N * RS // SC_WIN
    nblk = N * RS // SC_WIN

    @functools.partial(
        pl.kernel,
        out_type=jax.ShapeDtypeStruct((NROW * RS, DS), jnp.float32),
        mesh=mesh,
        scratch_types=[],
    )
    def k(x_hbm, i_hbm, o_hbm):
        def body(x_vmem, i_vmem):
            pltpu.sync_copy(x_vmem, o_hbm.at[i_vmem.at[0]])

        pltpu.emit_pipeline(
            body,
            grid=(nwin,),
            in_specs=[
                pl.BlockSpec((SC_WIN, DS), index_map=lambda i: (i % nblk, 0)),
                pl.BlockSpec((1, SC_WIN), index_map=lambda i: (i, 0)),
            ],
            out_specs=[],
            core_axis_name=("c", "s"),
            dimension_semantics=(pltpu.PARALLEL,),
        )(x_hbm, i_hbm)

    return k(x4, idx)


def _mlp_kernel(meta_ref, xs_ref, wg_ref, wu_ref, wd_ref, out_ref):
    j = pl.program_id(0)
    c = pl.program_id(1)
    nt_used = meta_ref[NT_EXP]

    @pl.when(j < nt_used)
    def _():
        x = xs_ref[...].astype(jnp.bfloat16)
        wg = wg_ref[0].astype(jnp.bfloat16)
        wu = wu_ref[0].astype(jnp.bfloat16)
        wd = wd_ref[0].astype(jnp.bfloat16)
        g = jax.lax.dot_general(
            x, wg, (((1,), (1,)), ((), ())), preferred_element_type=jnp.float32)
        u = jax.lax.dot_general(
            x, wu, (((1,), (1,)), ((), ())), preferred_element_type=jnp.float32)
        h = (g * jax.lax.logistic(g) * u).astype(jnp.bfloat16)
        part = jax.lax.dot_general(
            h, wd, (((1,), (1,)), ((), ())), preferred_element_type=jnp.float32)

        @pl.when(c == 0)
        def _():
            out_ref[...] = part

        @pl.when(c > 0)
        def _():
            out_ref[...] += part


def _grouped_mlp(meta, xs, wg, wu, wd):
    grid_spec = pltpu.PrefetchScalarGridSpec(
        num_scalar_prefetch=1,
        grid=(NT_EXP, NC),
        in_specs=[
            pl.BlockSpec((TM, D), lambda j, c, meta: (j, 0)),
            pl.BlockSpec((1, IC, D), lambda j, c, meta: (meta[j], c, 0)),
            pl.BlockSpec((1, IC, D), lambda j, c, meta: (meta[j], c, 0)),
            pl.BlockSpec((1, D, IC), lambda j, c, meta: (meta[j], 0, c)),
        ],
        out_specs=pl.BlockSpec((TM, D), lambda j, c, meta: (j, 0)),
    )
    return pl.pallas_call(
        _mlp_kernel,
        grid_spec=grid_spec,
        out_shape=jax.ShapeDtypeStruct((NROW, D), jnp.float32),
    )(meta, xs, wg, wu, wd)


def _shared_kernel(x_ref, wg_ref, wu_ref, wd_ref, out_ref):
    x = x_ref[...].astype(jnp.bfloat16)
    g = jax.lax.dot_general(
        x, wg_ref[...], (((1,), (1,)), ((), ())),
        preferred_element_type=jnp.float32)
    u = jax.lax.dot_general(
        x, wu_ref[...], (((1,), (1,)), ((), ())),
        preferred_element_type=jnp.float32)
    h = (g * jax.lax.logistic(g) * u).astype(jnp.bfloat16)
    out_ref[...] = jax.lax.dot_general(
        h, wd_ref[...], (((1,), (1,)), ((), ())),
        preferred_element_type=jnp.float32)


def _shared_mlp(x, swg, swu, swd):
    return pl.pallas_call(
        _shared_kernel,
        grid=(N // T,),
        in_specs=[
            pl.BlockSpec((T, D), lambda i: (i, 0)),
            pl.BlockSpec((II, D), lambda i: (0, 0)),
            pl.BlockSpec((II, D), lambda i: (0, 0)),
            pl.BlockSpec((D, II), lambda i: (0, 0)),
        ],
        out_specs=pl.BlockSpec((T, D), lambda i: (i, 0)),
        out_shape=jax.ShapeDtypeStruct((N, D), jnp.float32),
    )(x, swg, swu, swd)


def _combine_gather(ys4, idx):
    mesh = plsc.VectorSubcoreMesh(core_axis_name="c", subcore_axis_name="s")
    nwin = 2 * N * RS // SC_WIN

    @functools.partial(
        pl.kernel,
        out_type=jax.ShapeDtypeStruct((2 * N * RS, DS), jnp.float32),
        mesh=mesh,
        scratch_types=[],
    )
    def k(y_hbm, i_hbm, o_hbm):
        def body(i_vmem, o_vmem):
            pltpu.sync_copy(y_hbm.at[i_vmem.at[0]], o_vmem)

        pltpu.emit_pipeline(
            body,
            grid=(nwin,),
            in_specs=[pl.BlockSpec((1, SC_WIN), index_map=lambda i: (i, 0))],
            out_specs=[pl.BlockSpec((SC_WIN, DS), index_map=lambda i: (i, 0))],
            core_axis_name=("c", "s"),
            dimension_semantics=(pltpu.PARALLEL,),
        )(i_hbm, o_hbm)

    return k(ys4, idx)


def _combine_kernel(y0_ref, y1_ref, ysh_ref, w_ref, o_ref):
    w = w_ref[...]
    o_ref[...] = (y0_ref[...] * w[:, 0:1] + y1_ref[...] * w[:, 1:2]
                  + ysh_ref[...])


def _combine(y2, ysh, wts):
    nb = N // T
    return pl.pallas_call(
        _combine_kernel,
        grid=(nb,),
        in_specs=[
            pl.BlockSpec((T, D), lambda i: (i, 0)),
            pl.BlockSpec((T, D), lambda i: (i + nb, 0)),
            pl.BlockSpec((T, D), lambda i: (i, 0)),
            pl.BlockSpec((T, 2), lambda i: (i, 0)),
        ],
        out_specs=pl.BlockSpec((T, D), lambda i: (i, 0)),
        out_shape=jax.ShapeDtypeStruct((N, D), jnp.float32),
    )(y2, y2, ysh, wts)


def kernel(hidden_states, gate_W, Wg, Wu, Wd, sWg, sWu, sWd):
    b, l, d = hidden_states.shape
    x = hidden_states.reshape(l, d)
    swg_t = sWg.astype(jnp.bfloat16)
    swu_t = sWu.astype(jnp.bfloat16)
    swd_t = sWd.astype(jnp.bfloat16)
    slots, meta, wts, aux = _router(x, gate_W)
    idx = (slots.reshape(-1)[:, None] * RS
           + jnp.arange(RS, dtype=jnp.int32)[None, :]).reshape(-1, SC_WIN)
    xs4 = _dispatch(x.reshape(N * RS, DS), idx)
    ysh = _shared_mlp(x, swg_t, swu_t, swd_t)
    ys = _grouped_mlp(meta.reshape(-1), xs4.reshape(NROW, D), Wg, Wu, Wd)
    y2 = _combine_gather(ys.reshape(NROW * RS, DS), idx)
    out = _combine(y2.reshape(2 * N, D), ysh, wts)
    return out.reshape(b, l, d), aux.reshape(())


# speedup vs baseline: 1.9377x; 1.1811x over previous
import functools

import jax
import jax.numpy as jnp
from jax.experimental import pallas as pl
from jax.experimental.pallas import tpu as pltpu
from jax.experimental.pallas import tpu_sc as plsc

N = 2048
D = 1024
II = 2816
E = 8
T = 256
TM = 512
NT_EXP = 2 * N // TM + E - 1
NROW = NT_EXP * TM
NC = 2
IC = II // NC
RS = 4
DS = D // RS
SC_WIN = 128


def _router_kernel(x_ref, gw_ref, slots_ref, meta_ref, wts_ref, aux_ref):
    x = x_ref[...]
    gw = gw_ref[...]
    logits = jax.lax.dot_general(
        x, gw, (((1,), (1,)), ((), ())), preferred_element_type=jnp.float32)
    ids = jax.lax.broadcasted_iota(jnp.int32, (N, E), 1).astype(jnp.float32)
    m1 = jnp.max(logits, axis=1, keepdims=True)
    i1 = jnp.min(jnp.where(logits == m1, ids, float(E)), axis=1, keepdims=True)
    masked = jnp.where(ids == i1, -jnp.inf, logits)
    m2 = jnp.max(masked, axis=1, keepdims=True)
    i2 = jnp.min(jnp.where(masked == m2, ids, float(E)), axis=1, keepdims=True)
    e2 = jnp.exp(m2 - m1)
    w1 = 1.0 / (1.0 + e2)
    wts_ref[...] = jnp.concatenate([w1, e2 * w1], axis=1)
    p = jnp.exp(logits - m1)
    probs = p / jnp.sum(p, axis=1, keepdims=True)
    meanprob = jnp.mean(probs, axis=0, keepdims=True)
    oh1 = (ids == i1).astype(jnp.float32)
    oh2 = (ids == i2).astype(jnp.float32)
    counts = jnp.sum(oh1 + oh2, axis=0, keepdims=True)
    aux_ref[...] = (0.01 * E / N) * jnp.sum(counts * meanprob).reshape(1, 1)
    m = jnp.concatenate([oh1, oh2], axis=0)
    a = m
    s = 1
    while s < 2 * N:
        a = a + jnp.concatenate(
            [jnp.zeros((s, E), jnp.float32), a[: 2 * N - s]], axis=0)
        s *= 2
    r = jnp.sum((a - m) * m, axis=1, keepdims=True)
    tiles = jnp.floor((counts + float(TM - 1)) / float(TM))
    ct = tiles
    s = 1
    while s < E:
        ct = ct + jnp.concatenate(
            [jnp.zeros((1, s), jnp.float32), ct[:, : E - s]], axis=1)
        s *= 2
    base = float(TM) * (ct - tiles)
    slot = jnp.sum(m * base, axis=1, keepdims=True) + r
    slots_ref[...] = slot.astype(jnp.int32)
    jvec = jax.lax.broadcasted_iota(
        jnp.int32, (NT_EXP + 1, 1), 0).astype(jnp.float32)
    done = jnp.sum((jvec >= ct).astype(jnp.float32), axis=1, keepdims=True)
    e_tile = jnp.minimum(done, float(E - 1))
    meta_ref[...] = jnp.where(
        jvec == float(NT_EXP), jnp.sum(tiles), e_tile).astype(jnp.int32)


def _router(x, gate_w):
    return pl.pallas_call(
        _router_kernel,
        out_shape=[
            jax.ShapeDtypeStruct((2 * N, 1), jnp.int32),
            jax.ShapeDtypeStruct((NT_EXP + 1, 1), jnp.int32),
            jax.ShapeDtypeStruct((N, 2), jnp.float32),
            jax.ShapeDtypeStruct((1, 1), jnp.float32),
        ],
    )(x, gate_w)


def _dispatch(x4, idx):
    mesh = plsc.VectorSubcoreMesh(core_axis_name="c", subcore_axis_name="s")
    nwin = 2 * N * RS // SC_WIN
    nblk = N * RS // SC_WIN

    @functools.partial(
        pl.kernel,
        out_type=jax.ShapeDtypeStruct((NROW * RS, DS), jnp.float32),
        mesh=mesh,
        scratch_types=[],
    )
    def k(x_hbm, i_hbm, o_hbm):
        def body(x_vmem, i_vmem):
            pltpu.sync_copy(x_vmem, o_hbm.at[i_vmem.at[0]])

        pltpu.emit_pipeline(
            body,
            grid=(nwin,),
            in_specs=[
                pl.BlockSpec((SC_WIN, DS), index_map=lambda i: (i % nblk, 0)),
                pl.BlockSpec((1, SC_WIN), index_map=lambda i: (i, 0)),
            ],
            out_specs=[],
            core_axis_name=("c", "s"),
            dimension_semantics=(pltpu.PARALLEL,),
        )(x_hbm, i_hbm)

    return k(x4, idx)


def _mlp_kernel(meta_ref, xs_ref, wg_ref, wu_ref, wd_ref, out_ref):
    j = pl.program_id(0)
    c = pl.program_id(1)
    nt_used = meta_ref[NT_EXP]

    @pl.when(j < nt_used)
    def _():
        wg = wg_ref[0].astype(jnp.bfloat16)
        wu = wu_ref[0].astype(jnp.bfloat16)
        wd = wd_ref[0].astype(jnp.bfloat16)
        x = xs_ref[...].reshape(TM, D).astype(jnp.bfloat16)
        g = jax.lax.dot_general(
            x, wg, (((1,), (1,)), ((), ())), preferred_element_type=jnp.float32)
        u = jax.lax.dot_general(
            x, wu, (((1,), (1,)), ((), ())), preferred_element_type=jnp.float32)
        h = (g * jax.lax.logistic(g) * u).astype(jnp.bfloat16)
        part = jax.lax.dot_general(
            h, wd, (((1,), (1,)), ((), ())), preferred_element_type=jnp.float32)
        part = part.reshape(TM * RS, DS)

        @pl.when(c == 0)
        def _():
            out_ref[...] = part

        @pl.when(c > 0)
        def _():
            out_ref[...] += part


def _grouped_mlp(meta, xs, wg, wu, wd):
    grid_spec = pltpu.PrefetchScalarGridSpec(
        num_scalar_prefetch=1,
        grid=(NT_EXP, NC),
        in_specs=[
            pl.BlockSpec((TM * RS, DS), lambda j, c, meta: (j, 0)),
            pl.BlockSpec((1, IC, D), lambda j, c, meta: (meta[j], c, 0)),
            pl.BlockSpec((1, IC, D), lambda j, c, meta: (meta[j], c, 0)),
            pl.BlockSpec((1, D, IC), lambda j, c, meta: (meta[j], 0, c)),
        ],
        out_specs=pl.BlockSpec((TM * RS, DS), lambda j, c, meta: (j, 0)),
    )
    return pl.pallas_call(
        _mlp_kernel,
        grid_spec=grid_spec,
        out_shape=jax.ShapeDtypeStruct((NROW * RS, DS), jnp.float32),
    )(meta, xs, wg, wu, wd)


def _shared_kernel(x_ref, wg_ref, wu_ref, wd_ref, out_ref):
    x = x_ref[...].astype(jnp.bfloat16)
    g = jax.lax.dot_general(
        x, wg_ref[...], (((1,), (1,)), ((), ())),
        preferred_element_type=jnp.float32)
    u = jax.lax.dot_general(
        x, wu_ref[...], (((1,), (1,)), ((), ())),
        preferred_element_type=jnp.float32)
    h = (g * jax.lax.logistic(g) * u).astype(jnp.bfloat16)
    out_ref[...] = jax.lax.dot_general(
        h, wd_ref[...], (((1,), (1,)), ((), ())),
        preferred_element_type=jnp.float32)


def _shared_mlp(x, swg, swu, swd):
    return pl.pallas_call(
        _shared_kernel,
        grid=(N // T,),
        in_specs=[
            pl.BlockSpec((T, D), lambda i: (i, 0)),
            pl.BlockSpec((II, D), lambda i: (0, 0)),
            pl.BlockSpec((II, D), lambda i: (0, 0)),
            pl.BlockSpec((D, II), lambda i: (0, 0)),
        ],
        out_specs=pl.BlockSpec((T, D), lambda i: (i, 0)),
        out_shape=jax.ShapeDtypeStruct((N, D), jnp.float32),
    )(x, swg, swu, swd)


def _combine_gather(ys4, idx):
    mesh = plsc.VectorSubcoreMesh(core_axis_name="c", subcore_axis_name="s")
    nwin = 2 * N * RS // SC_WIN

    @functools.partial(
        pl.kernel,
        out_type=jax.ShapeDtypeStruct((2 * N * RS, DS), jnp.float32),
        mesh=mesh,
        scratch_types=[],
    )
    def k(y_hbm, i_hbm, o_hbm):
        def body(i_vmem, o_vmem):
            pltpu.sync_copy(y_hbm.at[i_vmem.at[0]], o_vmem)

        pltpu.emit_pipeline(
            body,
            grid=(nwin,),
            in_specs=[pl.BlockSpec((1, SC_WIN), index_map=lambda i: (i, 0))],
            out_specs=[pl.BlockSpec((SC_WIN, DS), index_map=lambda i: (i, 0))],
            core_axis_name=("c", "s"),
            dimension_semantics=(pltpu.PARALLEL,),
        )(i_hbm, o_hbm)

    return k(ys4, idx)


def _combine_kernel(y0_ref, y1_ref, ysh_ref, w_ref, o_ref):
    w = w_ref[...]
    o_ref[...] = (y0_ref[...] * w[:, 0:1] + y1_ref[...] * w[:, 1:2]
                  + ysh_ref[...])


def _combine(y2, ysh, wts):
    nb = N // T
    return pl.pallas_call(
        _combine_kernel,
        grid=(nb,),
        in_specs=[
            pl.BlockSpec((T, D), lambda i: (i, 0)),
            pl.BlockSpec((T, D), lambda i: (i + nb, 0)),
            pl.BlockSpec((T, D), lambda i: (i, 0)),
            pl.BlockSpec((T, 2), lambda i: (i, 0)),
        ],
        out_specs=pl.BlockSpec((T, D), lambda i: (i, 0)),
        out_shape=jax.ShapeDtypeStruct((N, D), jnp.float32),
    )(y2, y2, ysh, wts)


def kernel(hidden_states, gate_W, Wg, Wu, Wd, sWg, sWu, sWd):
    b, l, d = hidden_states.shape
    x = hidden_states.reshape(l, d)
    swg_t = sWg.astype(jnp.bfloat16)
    swu_t = sWu.astype(jnp.bfloat16)
    swd_t = sWd.astype(jnp.bfloat16)
    slots, meta, wts, aux = _router(x, gate_W)
    idx = (slots.reshape(-1)[:, None] * RS
           + jnp.arange(RS, dtype=jnp.int32)[None, :]).reshape(-1, SC_WIN)
    xs4 = _dispatch(x.reshape(N * RS, DS), idx)
    ysh = _shared_mlp(x, swg_t, swu_t, swd_t)
    ys4 = _grouped_mlp(meta.reshape(-1), xs4, Wg, Wu, Wd)
    y2 = _combine_gather(ys4, idx)
    out = _combine(y2.reshape(2 * N, D), ysh, wts)
    return out.reshape(b, l, d), aux.reshape(())


# speedup vs baseline: 2.0657x; 1.0661x over previous
import functools

import jax
import jax.numpy as jnp
from jax.experimental import pallas as pl
from jax.experimental.pallas import tpu as pltpu
from jax.experimental.pallas import tpu_sc as plsc

N = 2048
D = 1024
II = 2816
E = 8
T = 256
TM = 512
NT_EXP = 2 * N // TM + E - 1
NROW = NT_EXP * TM
NC = 2
IC = II // NC
RS = 4
DS = D // RS
SC_WIN = 128


def _router_kernel(x_ref, gw_ref, slots_ref, meta_ref, wts_ref, aux_ref,
                   x4_ref):
    x = x_ref[...]
    gw = gw_ref[...]
    x4_ref[...] = x.reshape(N * RS, DS)
    logits = jax.lax.dot_general(
        x, gw, (((1,), (1,)), ((), ())), preferred_element_type=jnp.float32)
    ids = jax.lax.broadcasted_iota(jnp.int32, (N, E), 1).astype(jnp.float32)
    m1 = jnp.max(logits, axis=1, keepdims=True)
    i1 = jnp.min(jnp.where(logits == m1, ids, float(E)), axis=1, keepdims=True)
    masked = jnp.where(ids == i1, -jnp.inf, logits)
    m2 = jnp.max(masked, axis=1, keepdims=True)
    i2 = jnp.min(jnp.where(masked == m2, ids, float(E)), axis=1, keepdims=True)
    e2 = jnp.exp(m2 - m1)
    w1 = 1.0 / (1.0 + e2)
    wts_ref[...] = jnp.concatenate([w1, e2 * w1], axis=1)
    p = jnp.exp(logits - m1)
    probs = p / jnp.sum(p, axis=1, keepdims=True)
    meanprob = jnp.mean(probs, axis=0, keepdims=True)
    oh1 = (ids == i1).astype(jnp.float32)
    oh2 = (ids == i2).astype(jnp.float32)
    counts = jnp.sum(oh1 + oh2, axis=0, keepdims=True)
    aux_ref[...] = (0.01 * E / N) * jnp.sum(counts * meanprob).reshape(1, 1)
    m = jnp.concatenate([oh1, oh2], axis=0)
    a = m
    s = 1
    while s < 2 * N:
        a = a + jnp.concatenate(
            [jnp.zeros((s, E), jnp.float32), a[: 2 * N - s]], axis=0)
        s *= 2
    r = jnp.sum((a - m) * m, axis=1, keepdims=True)
    tiles = jnp.floor((counts + float(TM - 1)) / float(TM))
    ct = tiles
    s = 1
    while s < E:
        ct = ct + jnp.concatenate(
            [jnp.zeros((1, s), jnp.float32), ct[:, : E - s]], axis=1)
        s *= 2
    base = float(TM) * (ct - tiles)
    slot = jnp.sum(m * base, axis=1, keepdims=True) + r
    slots_ref[...] = slot.astype(jnp.int32)
    jvec = jax.lax.broadcasted_iota(
        jnp.int32, (NT_EXP + 1, 1), 0).astype(jnp.float32)
    done = jnp.sum((jvec >= ct).astype(jnp.float32), axis=1, keepdims=True)
    e_tile = jnp.minimum(done, float(E - 1))
    meta_ref[...] = jnp.where(
        jvec == float(NT_EXP), jnp.sum(tiles), e_tile).astype(jnp.int32)


def _router(x, gate_w):
    return pl.pallas_call(
        _router_kernel,
        out_shape=[
            jax.ShapeDtypeStruct((2 * N, 1), jnp.int32),
            jax.ShapeDtypeStruct((NT_EXP + 1, 1), jnp.int32),
            jax.ShapeDtypeStruct((N, 2), jnp.float32),
            jax.ShapeDtypeStruct((1, 1), jnp.float32),
            jax.ShapeDtypeStruct((N * RS, DS), jnp.float32),
        ],
    )(x, gate_w)


def _dispatch(x4, idx):
    mesh = plsc.VectorSubcoreMesh(core_axis_name="c", subcore_axis_name="s")
    nwin = 2 * N * RS // SC_WIN
    nblk = N * RS // SC_WIN

    @functools.partial(
        pl.kernel,
        out_type=jax.ShapeDtypeStruct((NROW * RS, DS), jnp.float32),
        mesh=mesh,
        scratch_types=[],
    )
    def k(x_hbm, i_hbm, o_hbm):
        def body(x_vmem, i_vmem):
            pltpu.sync_copy(x_vmem, o_hbm.at[i_vmem.at[0]])

        pltpu.emit_pipeline(
            body,
            grid=(nwin,),
            in_specs=[
                pl.BlockSpec((SC_WIN, DS), index_map=lambda i: (i % nblk, 0)),
                pl.BlockSpec((1, SC_WIN), index_map=lambda i: (i, 0)),
            ],
            out_specs=[],
            core_axis_name=("c", "s"),
            dimension_semantics=(pltpu.PARALLEL,),
        )(x_hbm, i_hbm)

    return k(x4, idx)


def _mlp_kernel(meta_ref, xs_ref, wg_ref, wu_ref, wd_ref, out_ref):
    j = pl.program_id(0)
    c = pl.program_id(1)
    nt_used = meta_ref[NT_EXP]

    @pl.when(j < nt_used)
    def _():
        wg = wg_ref[0].astype(jnp.bfloat16)
        wu = wu_ref[0].astype(jnp.bfloat16)
        wd = wd_ref[0].astype(jnp.bfloat16)
        x = xs_ref[...].reshape(TM, D).astype(jnp.bfloat16)
        g = jax.lax.dot_general(
            x, wg, (((1,), (1,)), ((), ())), preferred_element_type=jnp.float32)
        u = jax.lax.dot_general(
            x, wu, (((1,), (1,)), ((), ())), preferred_element_type=jnp.float32)
        h = (g * jax.lax.logistic(g) * u).astype(jnp.bfloat16)
        part = jax.lax.dot_general(
            h, wd, (((1,), (1,)), ((), ())), preferred_element_type=jnp.float32)
        part = part.reshape(TM * RS, DS)

        @pl.when(c == 0)
        def _():
            out_ref[...] = part

        @pl.when(c > 0)
        def _():
            out_ref[...] += part


def _grouped_mlp(meta, xs, wg, wu, wd):
    grid_spec = pltpu.PrefetchScalarGridSpec(
        num_scalar_prefetch=1,
        grid=(NT_EXP, NC),
        in_specs=[
            pl.BlockSpec((TM * RS, DS), lambda j, c, meta: (j, 0)),
            pl.BlockSpec((1, IC, D), lambda j, c, meta: (meta[j], c, 0)),
            pl.BlockSpec((1, IC, D), lambda j, c, meta: (meta[j], c, 0)),
            pl.BlockSpec((1, D, IC), lambda j, c, meta: (meta[j], 0, c)),
        ],
        out_specs=pl.BlockSpec((TM * RS, DS), lambda j, c, meta: (j, 0)),
    )
    return pl.pallas_call(
        _mlp_kernel,
        grid_spec=grid_spec,
        out_shape=jax.ShapeDtypeStruct((NROW * RS, DS), jnp.float32),
    )(meta, xs, wg, wu, wd)


def _shared_kernel(x_ref, wg_ref, wu_ref, wd_ref, out_ref):
    x = x_ref[...].astype(jnp.bfloat16)
    g = jax.lax.dot_general(
        x, wg_ref[...], (((1,), (1,)), ((), ())),
        preferred_element_type=jnp.float32)
    u = jax.lax.dot_general(
        x, wu_ref[...], (((1,), (1,)), ((), ())),
        preferred_element_type=jnp.float32)
    h = (g * jax.lax.logistic(g) * u).astype(jnp.bfloat16)
    out_ref[...] = jax.lax.dot_general(
        h, wd_ref[...], (((1,), (1,)), ((), ())),
        preferred_element_type=jnp.float32)


def _shared_mlp(x, swg, swu, swd):
    return pl.pallas_call(
        _shared_kernel,
        grid=(N // T,),
        in_specs=[
            pl.BlockSpec((T, D), lambda i: (i, 0)),
            pl.BlockSpec((II, D), lambda i: (0, 0)),
            pl.BlockSpec((II, D), lambda i: (0, 0)),
            pl.BlockSpec((D, II), lambda i: (0, 0)),
        ],
        out_specs=pl.BlockSpec((T, D), lambda i: (i, 0)),
        out_shape=jax.ShapeDtypeStruct((N, D), jnp.float32),
    )(x, swg, swu, swd)


def _combine_gather(ys4, idx):
    mesh = plsc.VectorSubcoreMesh(core_axis_name="c", subcore_axis_name="s")
    nwin = 2 * N * RS // SC_WIN

    @functools.partial(
        pl.kernel,
        out_type=jax.ShapeDtypeStruct((2 * N * RS, DS), jnp.float32),
        mesh=mesh,
        scratch_types=[],
    )
    def k(y_hbm, i_hbm, o_hbm):
        def body(i_vmem, o_vmem):
            pltpu.sync_copy(y_hbm.at[i_vmem.at[0]], o_vmem)

        pltpu.emit_pipeline(
            body,
            grid=(nwin,),
            in_specs=[pl.BlockSpec((1, SC_WIN), index_map=lambda i: (i, 0))],
            out_specs=[pl.BlockSpec((SC_WIN, DS), index_map=lambda i: (i, 0))],
            core_axis_name=("c", "s"),
            dimension_semantics=(pltpu.PARALLEL,),
        )(i_hbm, o_hbm)

    return k(ys4, idx)


def _combine_kernel(y0_ref, y1_ref, ysh_ref, w_ref, o_ref):
    w = w_ref[...]
    y0 = y0_ref[...].reshape(T, D)
    y1 = y1_ref[...].reshape(T, D)
    o_ref[...] = y0 * w[:, 0:1] + y1 * w[:, 1:2] + ysh_ref[...]


def _combine(y2, ysh, wts):
    nb = N // T
    return pl.pallas_call(
        _combine_kernel,
        grid=(nb,),
        in_specs=[
            pl.BlockSpec((T * RS, DS), lambda i: (i, 0)),
            pl.BlockSpec((T * RS, DS), lambda i: (i + nb, 0)),
            pl.BlockSpec((T, D), lambda i: (i, 0)),
            pl.BlockSpec((T, 2), lambda i: (i, 0)),
        ],
        out_specs=pl.BlockSpec((T, D), lambda i: (i, 0)),
        out_shape=jax.ShapeDtypeStruct((N, D), jnp.float32),
    )(y2, y2, ysh, wts)


def kernel(hidden_states, gate_W, Wg, Wu, Wd, sWg, sWu, sWd):
    b, l, d = hidden_states.shape
    x = hidden_states.reshape(l, d)
    swg_t = sWg.astype(jnp.bfloat16)
    swu_t = sWu.astype(jnp.bfloat16)
    swd_t = sWd.astype(jnp.bfloat16)
    slots, meta, wts, aux, x4 = _router(x, gate_W)
    idx = (slots.reshape(-1)[:, None] * RS
           + jnp.arange(RS, dtype=jnp.int32)[None, :]).reshape(-1, SC_WIN)
    xs4 = _dispatch(x4, idx)
    ysh = _shared_mlp(x, swg_t, swu_t, swd_t)
    ys4 = _grouped_mlp(meta.reshape(-1), xs4, Wg, Wu, Wd)
    y2 = _combine_gather(ys4, idx)
    out = _combine(y2, ysh, wts)
    return out.reshape(b, l, d), aux.reshape(())


# speedup vs baseline: 2.0664x; 1.0003x over previous
import functools

import jax
import jax.numpy as jnp
from jax.experimental import pallas as pl
from jax.experimental.pallas import tpu as pltpu
from jax.experimental.pallas import tpu_sc as plsc

N = 2048
D = 1024
II = 2816
E = 8
T = 256
TM = 512
NT_EXP = 2 * N // TM + E - 1
NROW = NT_EXP * TM
NC = 2
IC = II // NC
RS = 4
DS = D // RS
SC_WIN = 128


def _router_kernel(x_ref, gw_ref, slots_ref, meta_ref, wts_ref, aux_ref,
                   x4_ref):
    x = x_ref[...]
    gw = gw_ref[...]
    x4_ref[...] = x.reshape(N * RS, DS)
    logits = jax.lax.dot_general(
        x, gw, (((1,), (1,)), ((), ())), preferred_element_type=jnp.float32)
    ids = jax.lax.broadcasted_iota(jnp.int32, (N, E), 1).astype(jnp.float32)
    m1 = jnp.max(logits, axis=1, keepdims=True)
    i1 = jnp.min(jnp.where(logits == m1, ids, float(E)), axis=1, keepdims=True)
    masked = jnp.where(ids == i1, -jnp.inf, logits)
    m2 = jnp.max(masked, axis=1, keepdims=True)
    i2 = jnp.min(jnp.where(masked == m2, ids, float(E)), axis=1, keepdims=True)
    e2 = jnp.exp(m2 - m1)
    w1 = 1.0 / (1.0 + e2)
    wts_ref[...] = jnp.concatenate([w1, e2 * w1], axis=1)
    p = jnp.exp(logits - m1)
    probs = p / jnp.sum(p, axis=1, keepdims=True)
    meanprob = jnp.mean(probs, axis=0, keepdims=True)
    oh1 = (ids == i1).astype(jnp.float32)
    oh2 = (ids == i2).astype(jnp.float32)
    counts = jnp.sum(oh1 + oh2, axis=0, keepdims=True)
    aux_ref[...] = (0.01 * E / N) * jnp.sum(counts * meanprob).reshape(1, 1)
    m = jnp.concatenate([oh1, oh2], axis=0)
    a = m
    s = 1
    while s < 2 * N:
        a = a + jnp.concatenate(
            [jnp.zeros((s, E), jnp.float32), a[: 2 * N - s]], axis=0)
        s *= 2
    r = jnp.sum((a - m) * m, axis=1, keepdims=True)
    tiles = jnp.floor((counts + float(TM - 1)) / float(TM))
    ct = tiles
    s = 1
    while s < E:
        ct = ct + jnp.concatenate(
            [jnp.zeros((1, s), jnp.float32), ct[:, : E - s]], axis=1)
        s *= 2
    base = float(TM) * (ct - tiles)
    slot = jnp.sum(m * base, axis=1, keepdims=True) + r
    slots_ref[...] = slot.astype(jnp.int32)
    jvec = jax.lax.broadcasted_iota(
        jnp.int32, (NT_EXP + 1, 1), 0).astype(jnp.float32)
    done = jnp.sum((jvec >= ct).astype(jnp.float32), axis=1, keepdims=True)
    e_tile = jnp.minimum(done, float(E - 1))
    meta_ref[...] = jnp.where(
        jvec == float(NT_EXP), jnp.sum(tiles), e_tile).astype(jnp.int32)


def _router(x, gate_w):
    return pl.pallas_call(
        _router_kernel,
        out_shape=[
            jax.ShapeDtypeStruct((2 * N, 1), jnp.int32),
            jax.ShapeDtypeStruct((NT_EXP + 1, 1), jnp.int32),
            jax.ShapeDtypeStruct((N, 2), jnp.float32),
            jax.ShapeDtypeStruct((1, 1), jnp.float32),
            jax.ShapeDtypeStruct((N * RS, DS), jnp.float32),
        ],
    )(x, gate_w)


def _dispatch(x4, idx):
    mesh = plsc.VectorSubcoreMesh(core_axis_name="c", subcore_axis_name="s")
    nwin = 2 * N * RS // SC_WIN
    nblk = N * RS // SC_WIN

    @functools.partial(
        pl.kernel,
        out_type=jax.ShapeDtypeStruct((NROW * RS, DS), jnp.float32),
        mesh=mesh,
        scratch_types=[],
    )
    def k(x_hbm, i_hbm, o_hbm):
        def body(x_vmem, i_vmem):
            pltpu.sync_copy(x_vmem, o_hbm.at[i_vmem.at[0]])

        pltpu.emit_pipeline(
            body,
            grid=(nwin,),
            in_specs=[
                pl.BlockSpec((SC_WIN, DS), index_map=lambda i: (i % nblk, 0)),
                pl.BlockSpec((1, SC_WIN), index_map=lambda i: (i, 0)),
            ],
            out_specs=[],
            core_axis_name=("c", "s"),
            dimension_semantics=(pltpu.PARALLEL,),
        )(x_hbm, i_hbm)

    return k(x4, idx)


def _mlp_kernel(meta_ref, xs_ref, wg_ref, wu_ref, wd_ref, out_ref):
    j = pl.program_id(0)
    c = pl.program_id(1)
    nt_used = meta_ref[NT_EXP]

    @pl.when(j < nt_used)
    def _():
        wg = wg_ref[0].astype(jnp.bfloat16)
        wu = wu_ref[0].astype(jnp.bfloat16)
        wd = wd_ref[0].astype(jnp.bfloat16)
        x = xs_ref[...].reshape(TM, D).astype(jnp.bfloat16)
        g = jax.lax.dot_general(
            x, wg, (((1,), (1,)), ((), ())), preferred_element_type=jnp.float32)
        u = jax.lax.dot_general(
            x, wu, (((1,), (1,)), ((), ())), preferred_element_type=jnp.float32)
        h = (g * jax.lax.logistic(g) * u).astype(jnp.bfloat16)
        part = jax.lax.dot_general(
            h, wd, (((1,), (1,)), ((), ())), preferred_element_type=jnp.float32)
        part = part.reshape(TM * RS, DS)

        @pl.when(c == 0)
        def _():
            out_ref[...] = part

        @pl.when(c > 0)
        def _():
            out_ref[...] += part


def _grouped_mlp(meta, xs, wg, wu, wd):
    grid_spec = pltpu.PrefetchScalarGridSpec(
        num_scalar_prefetch=1,
        grid=(NT_EXP, NC),
        in_specs=[
            pl.BlockSpec((TM * RS, DS), lambda j, c, meta: (j, 0)),
            pl.BlockSpec((1, IC, D), lambda j, c, meta: (meta[j], c, 0)),
            pl.BlockSpec((1, IC, D), lambda j, c, meta: (meta[j], c, 0)),
            pl.BlockSpec((1, D, IC), lambda j, c, meta: (meta[j], 0, c)),
        ],
        out_specs=pl.BlockSpec((TM * RS, DS), lambda j, c, meta: (j, 0)),
    )
    return pl.pallas_call(
        _mlp_kernel,
        grid_spec=grid_spec,
        out_shape=jax.ShapeDtypeStruct((NROW * RS, DS), jnp.float32),
    )(meta, xs, wg, wu, wd)


def _shared_kernel(x_ref, wg_ref, wu_ref, wd_ref, out_ref):
    x = x_ref[...].astype(jnp.bfloat16)
    g = jax.lax.dot_general(
        x, wg_ref[...], (((1,), (1,)), ((), ())),
        preferred_element_type=jnp.float32)
    u = jax.lax.dot_general(
        x, wu_ref[...], (((1,), (1,)), ((), ())),
        preferred_element_type=jnp.float32)
    h = (g * jax.lax.logistic(g) * u).astype(jnp.bfloat16)
    out_ref[...] = jax.lax.dot_general(
        h, wd_ref[...], (((1,), (1,)), ((), ())),
        preferred_element_type=jnp.float32)


def _shared_mlp(x, swg, swu, swd):
    return pl.pallas_call(
        _shared_kernel,
        grid=(N // T,),
        in_specs=[
            pl.BlockSpec((T, D), lambda i: (i, 0)),
            pl.BlockSpec((II, D), lambda i: (0, 0)),
            pl.BlockSpec((II, D), lambda i: (0, 0)),
            pl.BlockSpec((D, II), lambda i: (0, 0)),
        ],
        out_specs=pl.BlockSpec((T, D), lambda i: (i, 0)),
        out_shape=jax.ShapeDtypeStruct((N, D), jnp.float32),
    )(x, swg, swu, swd)


def _combine_gather(ys4, idx):
    mesh = plsc.VectorSubcoreMesh(core_axis_name="c", subcore_axis_name="s")
    nwin = 2 * N * RS // SC_WIN

    @functools.partial(
        pl.kernel,
        out_type=jax.ShapeDtypeStruct((2 * N * RS, DS), jnp.float32),
        mesh=mesh,
        scratch_types=[],
    )
    def k(y_hbm, i_hbm, o_hbm):
        def body(i_vmem, o_vmem):
            pltpu.sync_copy(y_hbm.at[i_vmem.at[0]], o_vmem)

        pltpu.emit_pipeline(
            body,
            grid=(nwin,),
            in_specs=[pl.BlockSpec((1, SC_WIN), index_map=lambda i: (i, 0))],
            out_specs=[pl.BlockSpec((SC_WIN, DS), index_map=lambda i: (i, 0))],
            core_axis_name=("c", "s"),
            dimension_semantics=(pltpu.PARALLEL,),
        )(i_hbm, o_hbm)

    return k(ys4, idx)


def _combine_kernel(y0_ref, y1_ref, ysh_ref, w_ref, o_ref):
    w = w_ref[...]
    y0 = y0_ref[...].reshape(T, D)
    y1 = y1_ref[...].reshape(T, D)
    o_ref[...] = y0 * w[:, 0:1] + y1 * w[:, 1:2] + ysh_ref[...]


def _combine(y2, ysh, wts):
    nb = N // T
    return pl.pallas_call(
        _combine_kernel,
        grid=(nb,),
        in_specs=[
            pl.BlockSpec((T * RS, DS), lambda i: (i, 0)),
            pl.BlockSpec((T * RS, DS), lambda i: (i + nb, 0)),
            pl.BlockSpec((T, D), lambda i: (i, 0)),
            pl.BlockSpec((T, 2), lambda i: (i, 0)),
        ],
        out_specs=pl.BlockSpec((T, D), lambda i: (i, 0)),
        out_shape=jax.ShapeDtypeStruct((N, D), jnp.float32),
    )(y2, y2, ysh, wts)


def kernel(hidden_states, gate_W, Wg, Wu, Wd, sWg, sWu, sWd):
    b, l, d = hidden_states.shape
    x = hidden_states.reshape(l, d)
    swg_t = sWg.astype(jnp.bfloat16)
    swu_t = sWu.astype(jnp.bfloat16)
    swd_t = sWd.astype(jnp.bfloat16)
    ysh = _shared_mlp(x, swg_t, swu_t, swd_t)
    slots, meta, wts, aux, x4 = _router(x, gate_W)
    idx = (slots.reshape(-1)[:, None] * RS
           + jnp.arange(RS, dtype=jnp.int32)[None, :]).reshape(-1, SC_WIN)
    xs4 = _dispatch(x4, idx)
    ys4 = _grouped_mlp(meta.reshape(-1), xs4, Wg, Wu, Wd)
    y2 = _combine_gather(ys4, idx)
    out = _combine(y2, ysh, wts)
    return out.reshape(b, l, d), aux.reshape(())
